# baseline (speedup 1.0000x reference)
import functools

import jax
import jax.numpy as jnp
from jax import lax
from jax.experimental import pallas as pl
from jax.experimental.pallas import tpu as pltpu
from jax.experimental.pallas import tpu_sc as plsc

N = 10000
E = 160000
D = 256
H = 512
G = 16

NPAD = 10240
EPAD = 163840
CH = 128
C = H // CH
SCH = 64
SC = H // SCH
NC = 2
NS = 16
EROWS = EPAD // 128

_mesh = plsc.VectorSubcoreMesh(core_axis_name="c", subcore_axis_name="s")


@functools.partial(
    pl.kernel,
    mesh=_mesh,
    out_type=jax.ShapeDtypeStruct((NC * NPAD,), jnp.float32),
    scratch_types=[
        pltpu.VMEM((40, 128), jnp.int32),
        pltpu.VMEM((128,), jnp.float32),
        pltpu.VMEM((640,), jnp.float32),
        pltpu.VMEM_SHARED((NPAD,), jnp.float32),
    ],
)
def _deg_kernel(dst_hbm, out_hbm, idx_v, ones_v, stage_v, acc_sh):
    ci = lax.axis_index("c")
    si = lax.axis_index("s")

    for j in range(8):
        ones_v[pl.ds(j * 16, 16)] = jnp.ones((16,), jnp.float32)

    def zbody(j, carry):
        stage_v[pl.ds(j * 16, 16)] = jnp.zeros((16,), jnp.float32)
        return carry

    lax.fori_loop(0, 40, zbody, 0)
    pltpu.sync_copy(stage_v, acc_sh.at[pl.ds(si * 640, 640)])
    plsc.subcore_barrier()

    pltpu.sync_copy(dst_hbm.at[pl.ds(ci * 640 + si * 40, 40)], idx_v)

    def body(b, carry):
        pltpu.sync_copy(ones_v, acc_sh.at[idx_v.at[b]], add=True)
        return carry

    lax.fori_loop(0, 40, body, 0)
    plsc.subcore_barrier()
    pltpu.sync_copy(
        acc_sh.at[pl.ds(si * 640, 640)],
        out_hbm.at[pl.ds(ci * NPAD + si * 640, 640)],
    )


@functools.partial(
    pl.kernel,
    mesh=_mesh,
    compiler_params=pltpu.CompilerParams(use_tc_tiling_on_sc=False),
    out_type=jax.ShapeDtypeStruct((SC * NPAD, SCH), jnp.float32),
    scratch_types=[
        pltpu.VMEM((80, 128), jnp.int32),
        pltpu.VMEM((80, 128), jnp.int32),
        pltpu.VMEM((4, 128, SCH), jnp.float32),
        pltpu.VMEM((40, SCH), jnp.float32),
        pltpu.VMEM_SHARED((NPAD, SCH), jnp.float32),
        pltpu.SemaphoreType.DMA((4,)),
        pltpu.SemaphoreType.DMA((4,)),
    ],
)
def _scatter_kernel(table_hbm, src_hbm, dst_hbm, out_hbm,
                    src_v, dst_v, rows_v, stage_v, acc_sh, sems, ssems):
    ci = lax.axis_index("c")
    si = lax.axis_index("s")

    def zbody(j, carry):
        r = j // 4
        k = j % 4
        stage_v[r, pl.ds(k * 16, 16)] = jnp.zeros((16,), jnp.float32)
        return carry

    lax.fori_loop(0, 160, zbody, 0)

    pltpu.sync_copy(dst_hbm.at[pl.ds(si * 80, 80)], dst_v)

    for c in range(SC // NC):
        cglob = ci * (SC // NC) + c

        for j in range(16):
            pltpu.sync_copy(stage_v, acc_sh.at[pl.ds(si * 640 + j * 40, 40)])
        plsc.subcore_barrier()

        pltpu.sync_copy(src_hbm.at[pl.ds(cglob * EROWS + si * 80, 80)], src_v)

        for p in range(3):
            pltpu.async_copy(table_hbm.at[src_v.at[p]], rows_v.at[p], sems.at[p])

        def body(b, carry):
            cur = lax.rem(b, 4)
            nxt = lax.rem(b + 3, 4)

            pltpu.make_async_copy(
                table_hbm.at[src_v.at[b]], rows_v.at[cur], sems.at[cur]).wait()
            pltpu.async_copy(
                rows_v.at[cur], acc_sh.at[dst_v.at[b]], ssems.at[cur], add=True)

            @pl.when(jnp.logical_and(b >= 1, b < 77))
            def _():
                pltpu.make_async_copy(
                    rows_v.at[nxt], acc_sh.at[dst_v.at[b - 1]], ssems.at[nxt]
                ).wait()

            @pl.when(b < 77)
            def _():
                pltpu.async_copy(
                    table_hbm.at[src_v.at[b + 3]], rows_v.at[nxt], sems.at[nxt])

            return carry

        lax.fori_loop(0, 80, body, 0)

        for p in range(4):
            pltpu.make_async_copy(
                rows_v.at[p], acc_sh.at[dst_v.at[76 + p]], ssems.at[p]).wait()
        plsc.subcore_barrier()

        pltpu.sync_copy(
            acc_sh.at[pl.ds(si * 640, 640)],
            out_hbm.at[pl.ds(cglob * NPAD + si * 640, 640)],
        )
        plsc.subcore_barrier()


def _tc1_body(x_ref, w_ref, deg_ref, out_ref):
    d = lax.rsqrt(deg_ref[0, :] + deg_ref[1, :] + 1.0)[:, None]
    out_ref[...] = jnp.dot(
        x_ref[...], w_ref[...], preferred_element_type=jnp.float32) * d


def _tc1(x_pad, w1, deg2):
    return pl.pallas_call(
        _tc1_body,
        grid=(NPAD // 1024, C),
        in_specs=[
            pl.BlockSpec((1024, D), lambda i, c: (i, 0)),
            pl.BlockSpec((D, CH), lambda i, c: (0, c)),
            pl.BlockSpec((NC, 1024), lambda i, c: (0, i)),
        ],
        out_specs=pl.BlockSpec((1024, CH), lambda i, c: (c * (NPAD // 1024) + i, 0)),
        out_shape=jax.ShapeDtypeStruct((C * NPAD, CH), jnp.float32),
    )(x_pad, w1, deg2)


def _tc2_body(s_ref, h_ref, deg_ref, b_ref, w_ref, out_ref, acc_ref):
    k = pl.program_id(2)
    d = lax.rsqrt(deg_ref[0, :] + deg_ref[1, :] + 1.0)[:, None]
    s = jnp.concatenate([s_ref[0], s_ref[1]], axis=1)
    z = jnp.maximum(d * (s + h_ref[...]) + b_ref[0], 0.0)
    partial = jnp.dot(z, w_ref[...], preferred_element_type=jnp.float32)

    @pl.when(k == 0)
    def _():
        acc_ref[...] = partial

    @pl.when(k > 0)
    def _():
        acc_ref[...] += partial

    @pl.when(k == C - 1)
    def _():
        out_ref[...] = acc_ref[...] * d


def _tc2(s1, h1t, deg2, b1r, w2):
    nb = NPAD // 1024
    return pl.pallas_call(
        _tc2_body,
        grid=(nb, C, C),
        in_specs=[
            pl.BlockSpec((2, 1024, SCH), lambda i, c2, k: (k, i, 0)),
            pl.BlockSpec((1024, CH), lambda i, c2, k: (k * nb + i, 0)),
            pl.BlockSpec((NC, 1024), lambda i, c2, k: (0, i)),
            pl.BlockSpec((1, 1, CH), lambda i, c2, k: (k, 0, 0)),
            pl.BlockSpec((CH, CH), lambda i, c2, k: (k, c2)),
        ],
        out_specs=pl.BlockSpec((1024, CH), lambda i, c2, k: (c2 * nb + i, 0)),
        out_shape=jax.ShapeDtypeStruct((C * NPAD, CH), jnp.float32),
        scratch_shapes=[pltpu.VMEM((1024, CH), jnp.float32)],
    )(s1, h1t, deg2, b1r, w2)


def _tc3_body(s_ref, h_ref, deg_ref, b_ref, batch_ref, node_ref, graph_ref,
              acc_ref, cnt_ref):
    i = pl.program_id(1)
    d = lax.rsqrt(deg_ref[0, :] + deg_ref[1, :] + 1.0)[:, None]
    s = jnp.concatenate([s_ref[0], s_ref[1]], axis=1)
    ne = d * (s + h_ref[...]) + b_ref[0]
    node_ref[...] = ne

    bt = batch_ref[0]
    oh = (lax.broadcasted_iota(jnp.int32, (G, 1024), 0) == bt).astype(jnp.float32)
    psum = jnp.dot(oh, ne, preferred_element_type=jnp.float32)
    pcnt = jnp.sum(oh, axis=1, keepdims=True)

    @pl.when(i == 0)
    def _():
        acc_ref[...] = psum
        cnt_ref[...] = pcnt

    @pl.when(i > 0)
    def _():
        acc_ref[...] += psum
        cnt_ref[...] += pcnt

    @pl.when(i == (NPAD // 1024) - 1)
    def _():
        graph_ref[...] = acc_ref[...] / jnp.maximum(cnt_ref[...], 1.0)


def _tc3(s2, h2t, deg2, b2r, batch_rows):
    nb = NPAD // 1024
    return pl.pallas_call(
        _tc3_body,
        grid=(C, nb),
        in_specs=[
            pl.BlockSpec((2, 1024, SCH), lambda c, i: (c, i, 0)),
            pl.BlockSpec((1024, CH), lambda c, i: (c * nb + i, 0)),
            pl.BlockSpec((NC, 1024), lambda c, i: (0, i)),
            pl.BlockSpec((1, 1, CH), lambda c, i: (c, 0, 0)),
            pl.BlockSpec((1, 1, 1024), lambda c, i: (i, 0, 0)),
        ],
        out_specs=[
            pl.BlockSpec((1024, CH), lambda c, i: (i, c)),
            pl.BlockSpec((G, CH), lambda c, i: (0, c)),
        ],
        out_shape=[
            jax.ShapeDtypeStruct((NPAD, H), jnp.float32),
            jax.ShapeDtypeStruct((G, H), jnp.float32),
        ],
        scratch_shapes=[
            pltpu.VMEM((G, CH), jnp.float32),
            pltpu.VMEM((G, 1), jnp.float32),
        ],
    )(s2, h2t, deg2, b2r, batch_rows)


def kernel(x, edge_index, batch, W1, b1, W2, b2):
    edge32 = edge_index.astype(jnp.int32)
    src = edge32[0]
    dst = edge32[1]
    pad = EPAD - E
    srcp = jnp.concatenate([src, jnp.zeros((pad,), jnp.int32)])
    dstp = jnp.concatenate([dst, jnp.full((pad,), N, jnp.int32)])
    dst_rows = dstp.reshape(EROWS, 128)
    c8 = jnp.arange(SC, dtype=jnp.int32)
    view_off = ((c8 // 2) * 2 * NPAD + (c8 % 2))[:, None]
    src_rows = (2 * srcp[None, :] + view_off).reshape(SC * EROWS, 128)

    x_pad = jnp.concatenate([x, jnp.zeros((NPAD - N, D), jnp.float32)])
    batch_rows = jnp.concatenate(
        [batch.astype(jnp.int32), jnp.full((NPAD - N,), G, jnp.int32)]
    ).reshape(NPAD // 1024, 1, 1024)
    b1r = b1.reshape(C, 1, CH)
    b2r = b2.reshape(C, 1, CH)

    deg2 = _deg_kernel(dst_rows).reshape(NC, NPAD)

    h1t = _tc1(x_pad, W1, deg2)
    s1 = _scatter_kernel(h1t.reshape(SC * NPAD, SCH), src_rows, dst_rows)
    h2t = _tc2(s1.reshape(SC, NPAD, SCH), h1t, deg2, b1r, W2)
    s2 = _scatter_kernel(h2t.reshape(SC * NPAD, SCH), src_rows, dst_rows)
    node_full, graph_embedding = _tc3(
        s2.reshape(SC, NPAD, SCH), h2t, deg2, b2r, batch_rows)

    node_embeddings = node_full[:N]
    return (node_embeddings, graph_embedding)


# speedup vs baseline: 1.0044x; 1.0044x over previous
import functools

import jax
import jax.numpy as jnp
from jax import lax
from jax.experimental import pallas as pl
from jax.experimental.pallas import tpu as pltpu
from jax.experimental.pallas import tpu_sc as plsc

N = 10000
E = 160000
D = 256
H = 512
G = 16

NPAD = 10240
EPAD = 163840
CH = 128
C = H // CH
SCH = 64
SC = H // SCH
NC = 2
NS = 16
EROWS = EPAD // 128

_mesh = plsc.VectorSubcoreMesh(core_axis_name="c", subcore_axis_name="s")


@functools.partial(
    pl.kernel,
    mesh=_mesh,
    out_type=jax.ShapeDtypeStruct((NC * NPAD,), jnp.float32),
    scratch_types=[
        pltpu.VMEM((40, 128), jnp.int32),
        pltpu.VMEM((128,), jnp.float32),
        pltpu.VMEM((640,), jnp.float32),
        pltpu.VMEM_SHARED((NPAD,), jnp.float32),
    ],
)
def _deg_kernel(dst_hbm, out_hbm, idx_v, ones_v, stage_v, acc_sh):
    ci = lax.axis_index("c")
    si = lax.axis_index("s")

    for j in range(8):
        ones_v[pl.ds(j * 16, 16)] = jnp.ones((16,), jnp.float32)

    def zbody(j, carry):
        stage_v[pl.ds(j * 16, 16)] = jnp.zeros((16,), jnp.float32)
        return carry

    lax.fori_loop(0, 40, zbody, 0)
    pltpu.sync_copy(stage_v, acc_sh.at[pl.ds(si * 640, 640)])
    plsc.subcore_barrier()

    pltpu.sync_copy(dst_hbm.at[pl.ds(ci * 640 + si * 40, 40)], idx_v)

    def body(b, carry):
        pltpu.sync_copy(ones_v, acc_sh.at[idx_v.at[b]], add=True)
        return carry

    lax.fori_loop(0, 40, body, 0)
    plsc.subcore_barrier()
    pltpu.sync_copy(
        acc_sh.at[pl.ds(si * 640, 640)],
        out_hbm.at[pl.ds(ci * NPAD + si * 640, 640)],
    )


@functools.partial(
    pl.kernel,
    mesh=_mesh,
    out_type=jax.ShapeDtypeStruct((C * NPAD, CH), jnp.float32),
    scratch_types=[
        pltpu.VMEM((8, 128), jnp.int32),
        pltpu.VMEM((8, 128), jnp.int32),
        pltpu.VMEM((2, 128, CH), jnp.float32),
        pltpu.VMEM((8, CH), jnp.float32),
        pltpu.VMEM_SHARED((NPAD, CH), jnp.float32),
        pltpu.SemaphoreType.DMA((2,)),
    ],
)
def _scatter_kernel(table_hbm, src_hbm, dst_hbm, out_hbm,
                    src_v, dst_v, rows_v, stage_v, acc_sh, sems):
    ci = lax.axis_index("c")
    si = lax.axis_index("s")

    def zbody(j, carry):
        r = j // 8
        k = j % 8
        stage_v[r, pl.ds(k * 16, 16)] = jnp.zeros((16,), jnp.float32)
        return carry

    lax.fori_loop(0, 64, zbody, 0)

    for c in range(C // NC):
        cglob = ci * (C // NC) + c

        def zacc(j, carry):
            pltpu.sync_copy(stage_v, acc_sh.at[pl.ds(si * 640 + j * 8, 8)])
            return carry

        lax.fori_loop(0, 80, zacc, 0)
        plsc.subcore_barrier()

        def group(g, carry):
            pltpu.sync_copy(
                src_hbm.at[pl.ds(cglob * EROWS + si * 80 + g * 8, 8)], src_v)
            pltpu.sync_copy(dst_hbm.at[pl.ds(si * 80 + g * 8, 8)], dst_v)

            pltpu.async_copy(table_hbm.at[src_v.at[0]], rows_v.at[0], sems.at[0])

            def body(b, carry2):
                cur = lax.rem(b, 2)
                nxt = lax.rem(b + 1, 2)

                @pl.when(b < 7)
                def _():
                    pltpu.async_copy(
                        table_hbm.at[src_v.at[b + 1]], rows_v.at[nxt],
                        sems.at[nxt])

                pltpu.make_async_copy(
                    table_hbm.at[src_v.at[b]], rows_v.at[cur],
                    sems.at[cur]).wait()
                pltpu.sync_copy(
                    rows_v.at[cur], acc_sh.at[dst_v.at[b]], add=True)
                return carry2

            lax.fori_loop(0, 8, body, 0)
            return carry

        lax.fori_loop(0, 10, group, 0)
        plsc.subcore_barrier()

        pltpu.sync_copy(
            acc_sh.at[pl.ds(si * 640, 640)],
            out_hbm.at[pl.ds(cglob * NPAD + si * 640, 640)],
        )
        plsc.subcore_barrier()


def _tc1_body(x_ref, w_ref, deg_ref, out_ref):
    d = lax.rsqrt(deg_ref[0, :] + deg_ref[1, :] + 1.0)[:, None]
    out_ref[...] = jnp.dot(
        x_ref[...], w_ref[...], preferred_element_type=jnp.float32) * d


def _tc1(x_pad, w1, deg2):
    return pl.pallas_call(
        _tc1_body,
        grid=(NPAD // 1024, C),
        in_specs=[
            pl.BlockSpec((1024, D), lambda i, c: (i, 0)),
            pl.BlockSpec((D, CH), lambda i, c: (0, c)),
            pl.BlockSpec((NC, 1024), lambda i, c: (0, i)),
        ],
        out_specs=pl.BlockSpec((1024, CH), lambda i, c: (c * (NPAD // 1024) + i, 0)),
        out_shape=jax.ShapeDtypeStruct((C * NPAD, CH), jnp.float32),
    )(x_pad, w1, deg2)


def _tc2_body(s_ref, h_ref, deg_ref, b_ref, w_ref, out_ref, acc_ref):
    k = pl.program_id(2)
    d = lax.rsqrt(deg_ref[0, :] + deg_ref[1, :] + 1.0)[:, None]
    z = jnp.maximum(d * (s_ref[...] + h_ref[...]) + b_ref[0], 0.0)
    partial = jnp.dot(z, w_ref[...], preferred_element_type=jnp.float32)

    @pl.when(k == 0)
    def _():
        acc_ref[...] = partial

    @pl.when(k > 0)
    def _():
        acc_ref[...] += partial

    @pl.when(k == C - 1)
    def _():
        out_ref[...] = acc_ref[...] * d


def _tc2(s1, h1t, deg2, b1r, w2):
    nb = NPAD // 1024
    return pl.pallas_call(
        _tc2_body,
        grid=(nb, C, C),
        in_specs=[
            pl.BlockSpec((1024, CH), lambda i, c2, k: (k * nb + i, 0)),
            pl.BlockSpec((1024, CH), lambda i, c2, k: (k * nb + i, 0)),
            pl.BlockSpec((NC, 1024), lambda i, c2, k: (0, i)),
            pl.BlockSpec((1, 1, CH), lambda i, c2, k: (k, 0, 0)),
            pl.BlockSpec((CH, CH), lambda i, c2, k: (k, c2)),
        ],
        out_specs=pl.BlockSpec((1024, CH), lambda i, c2, k: (c2 * nb + i, 0)),
        out_shape=jax.ShapeDtypeStruct((C * NPAD, CH), jnp.float32),
        scratch_shapes=[pltpu.VMEM((1024, CH), jnp.float32)],
    )(s1, h1t, deg2, b1r, w2)


def _tc3_body(s_ref, h_ref, deg_ref, b_ref, batch_ref, node_ref, graph_ref,
              acc_ref, cnt_ref):
    i = pl.program_id(1)
    d = lax.rsqrt(deg_ref[0, :] + deg_ref[1, :] + 1.0)[:, None]
    ne = d * (s_ref[...] + h_ref[...]) + b_ref[0]
    node_ref[...] = ne

    bt = batch_ref[0]
    oh = (lax.broadcasted_iota(jnp.int32, (G, 1024), 0) == bt).astype(jnp.float32)
    psum = jnp.dot(oh, ne, preferred_element_type=jnp.float32)
    pcnt = jnp.sum(oh, axis=1, keepdims=True)

    @pl.when(i == 0)
    def _():
        acc_ref[...] = psum
        cnt_ref[...] = pcnt

    @pl.when(i > 0)
    def _():
        acc_ref[...] += psum
        cnt_ref[...] += pcnt

    @pl.when(i == (NPAD // 1024) - 1)
    def _():
        graph_ref[...] = acc_ref[...] / jnp.maximum(cnt_ref[...], 1.0)


def _tc3(s2, h2t, deg2, b2r, batch_rows):
    nb = NPAD // 1024
    return pl.pallas_call(
        _tc3_body,
        grid=(C, nb),
        in_specs=[
            pl.BlockSpec((1024, CH), lambda c, i: (c * nb + i, 0)),
            pl.BlockSpec((1024, CH), lambda c, i: (c * nb + i, 0)),
            pl.BlockSpec((NC, 1024), lambda c, i: (0, i)),
            pl.BlockSpec((1, 1, CH), lambda c, i: (c, 0, 0)),
            pl.BlockSpec((1, 1, 1024), lambda c, i: (i, 0, 0)),
        ],
        out_specs=[
            pl.BlockSpec((1024, CH), lambda c, i: (i, c)),
            pl.BlockSpec((G, CH), lambda c, i: (0, c)),
        ],
        out_shape=[
            jax.ShapeDtypeStruct((NPAD, H), jnp.float32),
            jax.ShapeDtypeStruct((G, H), jnp.float32),
        ],
        scratch_shapes=[
            pltpu.VMEM((G, CH), jnp.float32),
            pltpu.VMEM((G, 1), jnp.float32),
        ],
    )(s2, h2t, deg2, b2r, batch_rows)


def kernel(x, edge_index, batch, W1, b1, W2, b2):
    edge32 = edge_index.astype(jnp.int32)
    src = edge32[0]
    dst = edge32[1]
    pad = EPAD - E
    srcp = jnp.concatenate([src, jnp.zeros((pad,), jnp.int32)])
    dstp = jnp.concatenate([dst, jnp.full((pad,), N, jnp.int32)])
    dst_rows = dstp.reshape(EROWS, 128)
    chunk_off = (jnp.arange(C, dtype=jnp.int32) * NPAD)[:, None]
    src_rows = (srcp[None, :] + chunk_off).reshape(C * EROWS, 128)

    x_pad = jnp.concatenate([x, jnp.zeros((NPAD - N, D), jnp.float32)])
    batch_rows = jnp.concatenate(
        [batch.astype(jnp.int32), jnp.full((NPAD - N,), G, jnp.int32)]
    ).reshape(NPAD // 1024, 1, 1024)
    b1r = b1.reshape(C, 1, CH)
    b2r = b2.reshape(C, 1, CH)

    deg2 = _deg_kernel(dst_rows).reshape(NC, NPAD)

    h1t = _tc1(x_pad, W1, deg2)
    s1 = _scatter_kernel(h1t, src_rows, dst_rows)
    h2t = _tc2(s1, h1t, deg2, b1r, W2)
    s2 = _scatter_kernel(h2t, src_rows, dst_rows)
    node_full, graph_embedding = _tc3(s2, h2t, deg2, b2r, batch_rows)

    node_embeddings = node_full[:N]
    return (node_embeddings, graph_embedding)


# speedup vs baseline: 1.3879x; 1.3818x over previous
import functools

import jax
import jax.numpy as jnp
from jax import lax
from jax.experimental import pallas as pl
from jax.experimental.pallas import tpu as pltpu
from jax.experimental.pallas import tpu_sc as plsc

N = 10000
E = 160000
D = 256
H = 512
G = 16

NPAD = 10240
EPAD = 163840
CH = 128
C = H // CH
SCH = 64
SC = H // SCH
NC = 2
NS = 16
EROWS = EPAD // 128

_mesh = plsc.VectorSubcoreMesh(core_axis_name="c", subcore_axis_name="s")


@functools.partial(
    pl.kernel,
    mesh=_mesh,
    out_type=jax.ShapeDtypeStruct((NC * NPAD,), jnp.float32),
    scratch_types=[
        pltpu.VMEM((40, 128), jnp.int32),
        pltpu.VMEM((128,), jnp.float32),
        pltpu.VMEM((640,), jnp.float32),
        pltpu.VMEM_SHARED((NPAD,), jnp.float32),
    ],
)
def _deg_kernel(dst_hbm, out_hbm, idx_v, ones_v, stage_v, acc_sh):
    ci = lax.axis_index("c")
    si = lax.axis_index("s")

    for j in range(8):
        ones_v[pl.ds(j * 16, 16)] = jnp.ones((16,), jnp.float32)

    def zbody(j, carry):
        stage_v[pl.ds(j * 16, 16)] = jnp.zeros((16,), jnp.float32)
        return carry

    lax.fori_loop(0, 40, zbody, 0)
    pltpu.sync_copy(stage_v, acc_sh.at[pl.ds(si * 640, 640)])
    plsc.subcore_barrier()

    pltpu.sync_copy(dst_hbm.at[pl.ds(ci * 640 + si * 40, 40)], idx_v)

    def body(b, carry):
        pltpu.sync_copy(ones_v, acc_sh.at[idx_v.at[b]], add=True)
        return carry

    lax.fori_loop(0, 40, body, 0)
    plsc.subcore_barrier()
    pltpu.sync_copy(
        acc_sh.at[pl.ds(si * 640, 640)],
        out_hbm.at[pl.ds(ci * NPAD + si * 640, 640)],
    )


@functools.partial(
    pl.kernel,
    mesh=_mesh,
    compiler_params=pltpu.CompilerParams(use_tc_tiling_on_sc=False),
    out_type=jax.ShapeDtypeStruct((SC * NPAD, SCH), jnp.float32),
    scratch_types=[
        pltpu.VMEM((16, 128), jnp.int32),
        pltpu.VMEM((16, 128), jnp.int32),
        pltpu.VMEM((4, 128, SCH), jnp.float32),
        pltpu.VMEM((8, SCH), jnp.float32),
        pltpu.VMEM_SHARED((NPAD, SCH), jnp.float32),
        pltpu.VMEM_SHARED((NPAD, SCH), jnp.float32),
        pltpu.SemaphoreType.DMA((4,)),
    ],
)
def _scatter_kernel(table_hbm, src_hbm, dst_hbm, out_hbm,
                    src_v, dst_v, rows_v, stage_v, tbl_sh, acc_sh, sems):
    ci = lax.axis_index("c")
    si = lax.axis_index("s")

    def zbody(j, carry):
        r = j // 4
        k = j % 4
        stage_v[r, pl.ds(k * 16, 16)] = jnp.zeros((16,), jnp.float32)
        return carry

    lax.fori_loop(0, 32, zbody, 0)

    for c in range(SC // NC):
        cglob = ci * (SC // NC) + c

        pltpu.sync_copy(
            table_hbm.at[pl.ds(cglob * NPAD + si * 640, 640)],
            tbl_sh.at[pl.ds(si * 640, 640)],
        )

        def zacc(j, carry):
            pltpu.sync_copy(stage_v, acc_sh.at[pl.ds(si * 640 + j * 8, 8)])
            return carry

        lax.fori_loop(0, 80, zacc, 0)
        plsc.subcore_barrier()

        def group(g, carry):
            pltpu.sync_copy(
                src_hbm.at[pl.ds(si * 80 + g * 16, 16)], src_v)
            pltpu.sync_copy(dst_hbm.at[pl.ds(si * 80 + g * 16, 16)], dst_v)

            for p in range(3):
                pltpu.async_copy(
                    tbl_sh.at[src_v.at[p]], rows_v.at[p], sems.at[p])

            def body(b, carry2):
                cur = lax.rem(b, 4)
                nxt = lax.rem(b + 3, 4)

                @pl.when(b < 13)
                def _():
                    pltpu.async_copy(
                        tbl_sh.at[src_v.at[b + 3]], rows_v.at[nxt],
                        sems.at[nxt])

                pltpu.make_async_copy(
                    tbl_sh.at[src_v.at[b]], rows_v.at[cur],
                    sems.at[cur]).wait()
                pltpu.sync_copy(
                    rows_v.at[cur], acc_sh.at[dst_v.at[b]], add=True)
                return carry2

            lax.fori_loop(0, 16, body, 0)
            return carry

        lax.fori_loop(0, 5, group, 0)
        plsc.subcore_barrier()

        pltpu.sync_copy(
            acc_sh.at[pl.ds(si * 640, 640)],
            out_hbm.at[pl.ds(cglob * NPAD + si * 640, 640)],
        )
        plsc.subcore_barrier()


def _tc1_body(x_ref, w_ref, deg_ref, out_ref):
    d = lax.rsqrt(deg_ref[0, :] + deg_ref[1, :] + 1.0)[:, None]
    out_ref[...] = jnp.dot(
        x_ref[...], w_ref[...], preferred_element_type=jnp.float32) * d


def _tc1(x_pad, w1, deg2):
    return pl.pallas_call(
        _tc1_body,
        grid=(NPAD // 1024, C),
        in_specs=[
            pl.BlockSpec((1024, D), lambda i, c: (i, 0)),
            pl.BlockSpec((D, CH), lambda i, c: (0, c)),
            pl.BlockSpec((NC, 1024), lambda i, c: (0, i)),
        ],
        out_specs=pl.BlockSpec((1024, CH), lambda i, c: (c * (NPAD // 1024) + i, 0)),
        out_shape=jax.ShapeDtypeStruct((C * NPAD, CH), jnp.float32),
    )(x_pad, w1, deg2)


def _tc2_body(s_ref, h_ref, deg_ref, b_ref, w_ref, out_ref, acc_ref):
    k = pl.program_id(2)
    d = lax.rsqrt(deg_ref[0, :] + deg_ref[1, :] + 1.0)[:, None]
    s = jnp.concatenate([s_ref[0], s_ref[1]], axis=1)
    z = jnp.maximum(d * (s + h_ref[...]) + b_ref[0], 0.0)
    partial = jnp.dot(z, w_ref[...], preferred_element_type=jnp.float32)

    @pl.when(k == 0)
    def _():
        acc_ref[...] = partial

    @pl.when(k > 0)
    def _():
        acc_ref[...] += partial

    @pl.when(k == C - 1)
    def _():
        out_ref[...] = acc_ref[...] * d


def _tc2(s1, h1t, deg2, b1r, w2):
    nb = NPAD // 1024
    return pl.pallas_call(
        _tc2_body,
        grid=(nb, C, C),
        in_specs=[
            pl.BlockSpec((2, 1024, SCH), lambda i, c2, k: (k, i, 0)),
            pl.BlockSpec((1024, CH), lambda i, c2, k: (k * nb + i, 0)),
            pl.BlockSpec((NC, 1024), lambda i, c2, k: (0, i)),
            pl.BlockSpec((1, 1, CH), lambda i, c2, k: (k, 0, 0)),
            pl.BlockSpec((CH, CH), lambda i, c2, k: (k, c2)),
        ],
        out_specs=pl.BlockSpec((1024, CH), lambda i, c2, k: (c2 * nb + i, 0)),
        out_shape=jax.ShapeDtypeStruct((C * NPAD, CH), jnp.float32),
        scratch_shapes=[pltpu.VMEM((1024, CH), jnp.float32)],
    )(s1, h1t, deg2, b1r, w2)


def _tc3_body(s_ref, h_ref, deg_ref, b_ref, batch_ref, node_ref, graph_ref,
              acc_ref, cnt_ref):
    i = pl.program_id(1)
    d = lax.rsqrt(deg_ref[0, :] + deg_ref[1, :] + 1.0)[:, None]
    s = jnp.concatenate([s_ref[0], s_ref[1]], axis=1)
    ne = d * (s + h_ref[...]) + b_ref[0]
    node_ref[...] = ne

    bt = batch_ref[0]
    oh = (lax.broadcasted_iota(jnp.int32, (G, 1024), 0) == bt).astype(jnp.float32)
    psum = jnp.dot(oh, ne, preferred_element_type=jnp.float32)
    pcnt = jnp.sum(oh, axis=1, keepdims=True)

    @pl.when(i == 0)
    def _():
        acc_ref[...] = psum
        cnt_ref[...] = pcnt

    @pl.when(i > 0)
    def _():
        acc_ref[...] += psum
        cnt_ref[...] += pcnt

    @pl.when(i == (NPAD // 1024) - 1)
    def _():
        graph_ref[...] = acc_ref[...] / jnp.maximum(cnt_ref[...], 1.0)


def _tc3(s2, h2t, deg2, b2r, batch_rows):
    nb = NPAD // 1024
    return pl.pallas_call(
        _tc3_body,
        grid=(C, nb),
        in_specs=[
            pl.BlockSpec((2, 1024, SCH), lambda c, i: (c, i, 0)),
            pl.BlockSpec((1024, CH), lambda c, i: (c * nb + i, 0)),
            pl.BlockSpec((NC, 1024), lambda c, i: (0, i)),
            pl.BlockSpec((1, 1, CH), lambda c, i: (c, 0, 0)),
            pl.BlockSpec((1, 1, 1024), lambda c, i: (i, 0, 0)),
        ],
        out_specs=[
            pl.BlockSpec((1024, CH), lambda c, i: (i, c)),
            pl.BlockSpec((G, CH), lambda c, i: (0, c)),
        ],
        out_shape=[
            jax.ShapeDtypeStruct((NPAD, H), jnp.float32),
            jax.ShapeDtypeStruct((G, H), jnp.float32),
        ],
        scratch_shapes=[
            pltpu.VMEM((G, CH), jnp.float32),
            pltpu.VMEM((G, 1), jnp.float32),
        ],
    )(s2, h2t, deg2, b2r, batch_rows)


def kernel(x, edge_index, batch, W1, b1, W2, b2):
    edge32 = edge_index.astype(jnp.int32)
    src = edge32[0]
    dst = edge32[1]
    pad = EPAD - E
    srcp = jnp.concatenate([src, jnp.zeros((pad,), jnp.int32)])
    dstp = jnp.concatenate([dst, jnp.full((pad,), N, jnp.int32)])
    dst_rows = dstp.reshape(EROWS, 128)
    src_rows = srcp.reshape(EROWS, 128)

    x_pad = jnp.concatenate([x, jnp.zeros((NPAD - N, D), jnp.float32)])
    batch_rows = jnp.concatenate(
        [batch.astype(jnp.int32), jnp.full((NPAD - N,), G, jnp.int32)]
    ).reshape(NPAD // 1024, 1, 1024)
    b1r = b1.reshape(C, 1, CH)
    b2r = b2.reshape(C, 1, CH)

    deg2 = _deg_kernel(dst_rows).reshape(NC, NPAD)

    h1t = _tc1(x_pad, W1, deg2)
    s1 = _scatter_kernel(h1t.reshape(SC * NPAD, SCH), src_rows, dst_rows)
    h2t = _tc2(s1.reshape(SC, NPAD, SCH), h1t, deg2, b1r, W2)
    s2 = _scatter_kernel(h2t.reshape(SC * NPAD, SCH), src_rows, dst_rows)
    node_full, graph_embedding = _tc3(
        s2.reshape(SC, NPAD, SCH), h2t, deg2, b2r, batch_rows)

    node_embeddings = node_full[:N]
    return (node_embeddings, graph_embedding)
